# Initial kernel scaffold; baseline (speedup 1.0000x reference)
import jax
import jax.numpy as jnp
from jax.experimental import pallas as pl


def kernel(hidden_states, router_w, gate_up_proj, gate_up_proj_bias, down_proj, down_proj_bias):
    raise NotImplementedError("write your pallas kernel here")


# speedup vs baseline: 1.8086x; 1.8086x over previous
import functools

import jax
import jax.numpy as jnp
from jax.experimental import pallas as pl
from jax.experimental.pallas import tpu as pltpu

S = 2048
H = 768
E = 8
INTER = 768
LIMIT = 7.0
ALPHA = 1.702
TS = 256


def _moe_body(x_ref, rw_ref, wgu_ref, bgu_ref, wd_ref, bd_ref, out_ref):
    xb = x_ref[...]
    logits = jnp.dot(xb, rw_ref[...], preferred_element_type=jnp.float32)
    m = jnp.max(logits, axis=-1, keepdims=True)
    ex = jnp.exp(logits - m)
    scores = ex / jnp.sum(ex, axis=-1, keepdims=True)
    eidx = jax.lax.broadcasted_iota(jnp.int32, (TS, E), 1)
    a1 = jnp.min(jnp.where(logits == m, eidx, E), axis=-1, keepdims=True)
    sel1 = eidx == a1
    neg = jnp.float32(-jnp.inf)
    logits2 = jnp.where(sel1, neg, logits)
    m2 = jnp.max(logits2, axis=-1, keepdims=True)
    a2 = jnp.min(jnp.where(logits2 == m2, eidx, E), axis=-1, keepdims=True)
    sel2 = eidx == a2
    coeff = scores * jnp.where(sel1 | sel2, 1.0, 0.0)

    acc = jnp.zeros((TS, H), jnp.float32)
    for e in range(E):
        gu = jnp.dot(xb, wgu_ref[e], preferred_element_type=jnp.float32)
        gu = gu + bgu_ref[e][None, :]
        gate = jnp.minimum(gu[:, :INTER], LIMIT)
        up = jnp.clip(gu[:, INTER:], -LIMIT, LIMIT)
        glu = gate * jax.nn.sigmoid(gate * ALPHA)
        act = (up + 1.0) * glu
        y = jnp.dot(act.astype(jnp.bfloat16), wd_ref[e],
                    preferred_element_type=jnp.float32)
        acc = acc + coeff[:, e:e + 1] * (y + bd_ref[e][None, :])
    out_ref[...] = acc


@jax.jit
def _moe(x, rw, wgu, bgu, wd, bd):
    grid = (S // TS,)
    return pl.pallas_call(
        _moe_body,
        grid=grid,
        in_specs=[
            pl.BlockSpec((TS, H), lambda i: (i, 0)),
            pl.BlockSpec((H, E), lambda i: (0, 0)),
            pl.BlockSpec((E, H, 2 * INTER), lambda i: (0, 0, 0)),
            pl.BlockSpec((E, 2 * INTER), lambda i: (0, 0)),
            pl.BlockSpec((E, H, INTER), lambda i: (0, 0, 0)),
            pl.BlockSpec((E, H), lambda i: (0, 0)),
        ],
        out_specs=pl.BlockSpec((TS, H), lambda i: (i, 0)),
        out_shape=jax.ShapeDtypeStruct((S, H), jnp.float32),
        compiler_params=pltpu.CompilerParams(
            dimension_semantics=("arbitrary",),
        ),
    )(x, rw, wgu, bgu, wd, bd)


def kernel(hidden_states, router_w, gate_up_proj, gate_up_proj_bias, down_proj,
           down_proj_bias):
    b, s, h = hidden_states.shape
    x = hidden_states.reshape(s, h).astype(jnp.bfloat16)
    out = _moe(
        x,
        router_w.astype(jnp.bfloat16),
        gate_up_proj.astype(jnp.bfloat16),
        gate_up_proj_bias,
        down_proj.astype(jnp.bfloat16),
        down_proj_bias,
    )
    return out.reshape(b, s, h)


# speedup vs baseline: 1.8453x; 1.0203x over previous
import functools

import jax
import jax.numpy as jnp
from jax.experimental import pallas as pl
from jax.experimental.pallas import tpu as pltpu

S = 2048
H = 768
E = 8
INTER = 768
LIMIT = 7.0
ALPHA = 1.702
TS = 512


def _moe_body(x_ref, rw_ref, wgu_ref, bgu_ref, wd_ref, bd_ref, out_ref):
    xb = x_ref[...]
    logits = jnp.dot(xb, rw_ref[...], preferred_element_type=jnp.float32)
    m = jnp.max(logits, axis=-1, keepdims=True)
    ex = jnp.exp(logits - m)
    scores = ex / jnp.sum(ex, axis=-1, keepdims=True)
    eidx = jax.lax.broadcasted_iota(jnp.int32, (TS, E), 1)
    a1 = jnp.min(jnp.where(logits == m, eidx, E), axis=-1, keepdims=True)
    sel1 = eidx == a1
    neg = jnp.float32(-jnp.inf)
    logits2 = jnp.where(sel1, neg, logits)
    m2 = jnp.max(logits2, axis=-1, keepdims=True)
    a2 = jnp.min(jnp.where(logits2 == m2, eidx, E), axis=-1, keepdims=True)
    sel2 = eidx == a2
    coeff = scores * jnp.where(sel1 | sel2, 1.0, 0.0)

    acc = jnp.zeros((TS, H), jnp.float32)
    for e in range(E):
        gu = jnp.dot(xb, wgu_ref[e], preferred_element_type=jnp.float32)
        gu = gu + bgu_ref[e][None, :]
        gate = jnp.minimum(gu[:, :INTER], LIMIT)
        up = jnp.clip(gu[:, INTER:], -LIMIT, LIMIT)
        glu = gate * jax.nn.sigmoid(gate * ALPHA)
        act = (up + 1.0) * glu
        y = jnp.dot(act.astype(jnp.bfloat16), wd_ref[e],
                    preferred_element_type=jnp.float32)
        acc = acc + coeff[:, e:e + 1] * (y + bd_ref[e][None, :])
    out_ref[...] = acc


@jax.jit
def _moe(x, rw, wgu, bgu, wd, bd):
    grid = (S // TS,)
    return pl.pallas_call(
        _moe_body,
        grid=grid,
        in_specs=[
            pl.BlockSpec((TS, H), lambda i: (i, 0)),
            pl.BlockSpec((H, E), lambda i: (0, 0)),
            pl.BlockSpec((E, H, 2 * INTER), lambda i: (0, 0, 0)),
            pl.BlockSpec((E, 2 * INTER), lambda i: (0, 0)),
            pl.BlockSpec((E, H, INTER), lambda i: (0, 0, 0)),
            pl.BlockSpec((E, H), lambda i: (0, 0)),
        ],
        out_specs=pl.BlockSpec((TS, H), lambda i: (i, 0)),
        out_shape=jax.ShapeDtypeStruct((S, H), jnp.float32),
        compiler_params=pltpu.CompilerParams(
            dimension_semantics=("arbitrary",),
        ),
    )(x, rw, wgu, bgu, wd, bd)


def kernel(hidden_states, router_w, gate_up_proj, gate_up_proj_bias, down_proj,
           down_proj_bias):
    b, s, h = hidden_states.shape
    x = hidden_states.reshape(s, h).astype(jnp.bfloat16)
    out = _moe(
        x,
        router_w.astype(jnp.bfloat16),
        gate_up_proj.astype(jnp.bfloat16),
        gate_up_proj_bias,
        down_proj.astype(jnp.bfloat16),
        down_proj_bias,
    )
    return out.reshape(b, s, h)
